# baseline (speedup 1.0000x reference)
import functools

import jax
import jax.numpy as jnp
from jax import lax
from jax.experimental import pallas as pl
from jax.experimental.pallas import tpu as pltpu
from jax.experimental.pallas import tpu_sc as plsc

N = 1_000_000
B = 16_384
NC = 2
NS = 16
L = 16
NW = NC * NS
SHIFT = 15
CHUNK = 1 << SHIFT
FULL = N // CHUNK
LAST = N - FULL * CHUNK
STEPS = B // L
NSEG = 4
SEG = B // NSEG


def _scatter_body(idx_hbm, out_hbm, idx_v, buf_v, idx_sh, *sems):
    sid = lax.axis_index("s")
    wid = sid * NC + lax.axis_index("c")

    sh_copy = pltpu.make_async_copy(idx_hbm, idx_sh, sems[NSEG])

    @pl.when(sid == 0)
    def _():
        sh_copy.start()

    zeros = jnp.zeros((L,), jnp.int32)

    def zero_step(k, carry):
        buf_v[pl.ds(k * L, L)] = zeros
        return carry

    @pl.when(wid < FULL)
    def _():
        lax.fori_loop(0, CHUNK // L, zero_step, 0, unroll=16)

    @pl.when(wid == FULL)
    def _():
        lax.fori_loop(0, LAST // L, zero_step, 0, unroll=16)

    @pl.when(sid == 0)
    def _():
        sh_copy.wait()

    plsc.subcore_barrier()

    copies = []
    for k in range(NSEG):
        copies.append(
            pltpu.async_copy(
                idx_sh.at[pl.ds(k * SEG, SEG)],
                idx_v.at[pl.ds(k * SEG, SEG)],
                sems[k],
            )
        )

    iota = lax.iota(jnp.int32, L)
    spill_u = plsc.bitcast(iota + CHUNK, jnp.uint32)
    lo = wid * CHUNK

    def scan_step(step, carry):
        idxv = idx_v[pl.ds(step * L, L)]
        u = plsc.bitcast(idxv - lo, jnp.uint32)
        lidx = plsc.bitcast(jnp.minimum(u, spill_u), jnp.int32)
        plsc.store_scatter(buf_v, [lidx], iota + step * L)
        return carry

    seg_steps = SEG // L
    for k in range(NSEG):
        copies[k].wait()
        lax.fori_loop(
            k * seg_steps, (k + 1) * seg_steps, scan_step, 0, unroll=8
        )

    @pl.when(wid < FULL)
    def _():
        pltpu.sync_copy(
            buf_v.at[pl.ds(0, CHUNK)], out_hbm.at[pl.ds(wid * CHUNK, CHUNK)]
        )

    @pl.when(wid == FULL)
    def _():
        pltpu.sync_copy(
            buf_v.at[pl.ds(0, LAST)],
            out_hbm.at[pl.ds(FULL * CHUNK, LAST)],
        )


_scatter_kernel = functools.partial(
    pl.kernel,
    out_type=jax.ShapeDtypeStruct((N,), jnp.int32),
    mesh=plsc.VectorSubcoreMesh(
        core_axis_name="c", subcore_axis_name="s", num_cores=NC, num_subcores=NS
    ),
    scratch_types=[
        pltpu.VMEM((B,), jnp.int32),
        pltpu.VMEM((CHUNK + L,), jnp.int32),
        pltpu.VMEM_SHARED((B,), jnp.int32),
        pltpu.SemaphoreType.DMA,
        pltpu.SemaphoreType.DMA,
        pltpu.SemaphoreType.DMA,
        pltpu.SemaphoreType.DMA,
        pltpu.SemaphoreType.DMA,
    ],
    compiler_params=pltpu.CompilerParams(needs_layout_passes=False),
)(_scatter_body)


def kernel(X, row_missing_idx):
    del X
    return _scatter_kernel(row_missing_idx.astype(jnp.int32))


# speedup vs baseline: 1.0080x; 1.0080x over previous
import functools

import jax
import jax.numpy as jnp
from jax import lax
from jax.experimental import pallas as pl
from jax.experimental.pallas import tpu as pltpu
from jax.experimental.pallas import tpu_sc as plsc

N = 1_000_000
B = 16_384
NC = 2
NS = 16
L = 16
NW = NC * NS
SHIFT = 15
CHUNK = 1 << SHIFT
FULL = N // CHUNK
LAST = N - FULL * CHUNK
STEPS = B // L
NSEG = 4
SEG = B // NSEG


def _scatter_body(idx_hbm, out_hbm, idx_v, buf_v, idx_sh, *sems):
    sid = lax.axis_index("s")
    wid = sid * NC + lax.axis_index("c")

    sh_copy = pltpu.make_async_copy(idx_hbm, idx_sh, sems[NSEG])

    @pl.when(sid == 0)
    def _():
        sh_copy.start()

    zeros = jnp.zeros((L,), jnp.int32)

    def zero_step(k, carry):
        buf_v[pl.ds(k * L, L)] = zeros
        return carry

    @pl.when(wid < FULL)
    def _():
        lax.fori_loop(0, CHUNK // L, zero_step, 0, unroll=16)

    @pl.when(wid == FULL)
    def _():
        lax.fori_loop(0, LAST // L, zero_step, 0, unroll=16)

    @pl.when(sid == 0)
    def _():
        sh_copy.wait()

    plsc.subcore_barrier()

    copies = []
    for k in range(NSEG):
        copies.append(
            pltpu.async_copy(
                idx_sh.at[pl.ds(k * SEG, SEG)],
                idx_v.at[pl.ds(k * SEG, SEG)],
                sems[k],
            )
        )

    iota = lax.iota(jnp.int32, L)
    spill_u = plsc.bitcast(iota + CHUNK, jnp.uint32)
    lo = wid * CHUNK

    def scan_step(step, carry):
        idxv = idx_v[pl.ds(step * L, L)]
        u = plsc.bitcast(idxv - lo, jnp.uint32)
        lidx = plsc.bitcast(jnp.minimum(u, spill_u), jnp.int32)
        plsc.store_scatter(buf_v, [lidx], iota + step * L)
        return carry

    seg_steps = SEG // L
    for k in range(NSEG):
        copies[k].wait()
        lax.fori_loop(
            k * seg_steps, (k + 1) * seg_steps, scan_step, 0, unroll=4
        )

    @pl.when(wid < FULL)
    def _():
        pltpu.sync_copy(
            buf_v.at[pl.ds(0, CHUNK)], out_hbm.at[pl.ds(wid * CHUNK, CHUNK)]
        )

    @pl.when(wid == FULL)
    def _():
        pltpu.sync_copy(
            buf_v.at[pl.ds(0, LAST)],
            out_hbm.at[pl.ds(FULL * CHUNK, LAST)],
        )


_scatter_kernel = functools.partial(
    pl.kernel,
    out_type=jax.ShapeDtypeStruct((N,), jnp.int32),
    mesh=plsc.VectorSubcoreMesh(
        core_axis_name="c", subcore_axis_name="s", num_cores=NC, num_subcores=NS
    ),
    scratch_types=[
        pltpu.VMEM((B,), jnp.int32),
        pltpu.VMEM((CHUNK + L,), jnp.int32),
        pltpu.VMEM_SHARED((B,), jnp.int32),
        pltpu.SemaphoreType.DMA,
        pltpu.SemaphoreType.DMA,
        pltpu.SemaphoreType.DMA,
        pltpu.SemaphoreType.DMA,
        pltpu.SemaphoreType.DMA,
    ],
    compiler_params=pltpu.CompilerParams(needs_layout_passes=False),
)(_scatter_body)


def kernel(X, row_missing_idx):
    del X
    return _scatter_kernel(row_missing_idx.astype(jnp.int32))
